# Initial kernel scaffold; baseline (speedup 1.0000x reference)
import jax
import jax.numpy as jnp
from jax.experimental import pallas as pl


def kernel(features, adj_indices, adj_values, down_k, tokens, wp_weight, global_token, pre_token, combine_weight, balance_token, W1, b1, W2, b2):
    raise NotImplementedError("write your pallas kernel here")


# speedup vs baseline: 1.0057x; 1.0057x over previous
import functools

import jax
import jax.numpy as jnp
from jax.experimental import pallas as pl
from jax.experimental.pallas import tpu as pltpu

N_BLK = 256


def _prompt_body(f_ref, pt_ref, gt_ref, pre_ref, cw_ref, o_ref):
    f = f_ref[...]
    pt = pt_ref[...]
    x = jax.nn.relu(pt * f)
    x = gt_ref[...] * x
    x1 = pre_ref[...] * f
    hid = cw_ref[0, 0] * x + cw_ref[0, 1] * x1
    o_ref[...] = jnp.where(hid > 0, hid, jnp.exp(jnp.minimum(hid, 0.0)) - 1.0)


def _prompt_stage(features, pt, global_token, pre_token, combine_weight):
    n, h = features.shape
    grid = (pl.cdiv(n, N_BLK),)
    return pl.pallas_call(
        _prompt_body,
        grid=grid,
        in_specs=[
            pl.BlockSpec((N_BLK, h), lambda i: (i, 0)),
            pl.BlockSpec((1, h), lambda i: (0, 0)),
            pl.BlockSpec((1, h), lambda i: (0, 0)),
            pl.BlockSpec((1, h), lambda i: (0, 0)),
            pl.BlockSpec((1, 2), lambda i: (0, 0), memory_space=pltpu.SMEM),
        ],
        out_specs=pl.BlockSpec((N_BLK, h), lambda i: (i, 0)),
        out_shape=jax.ShapeDtypeStruct((n, h), jnp.float32),
    )(features, pt, global_token, pre_token, combine_weight)


def kernel(features, adj_indices, adj_values, down_k, tokens, wp_weight,
           global_token, pre_token, combine_weight, balance_token,
           W1, b1, W2, b2):
    n = features.shape[0]
    src = adj_indices[0]
    dst = adj_indices[1]
    pt = wp_weight @ tokens
    features1 = _prompt_stage(features, pt, global_token, pre_token,
                              combine_weight)

    agg = jax.ops.segment_sum(adj_values[:, None] * features1[src], dst,
                              num_segments=n)
    reseq1 = jnp.concatenate([features1, agg], axis=1)
    reseq111 = balance_token * reseq1
    z = reseq111 / (jnp.linalg.norm(reseq111, axis=1, keepdims=True) + 1e-8)
    sim = z @ z.T
    _, idx = jax.lax.top_k(sim, 10)
    vals = jnp.take_along_axis(sim, idx, axis=1)
    vals = jax.nn.relu(vals)
    vals = vals / (vals.sum(axis=1, keepdims=True) + 1e-8)
    alpha = 0.5

    def spmm_adj(h):
        return jax.ops.segment_sum(adj_values[:, None] * h[src], dst,
                                   num_segments=n)

    def spmm_re(h):
        return jnp.sum(vals[:, :, None] * h[idx], axis=1)

    def agg_new(h):
        return alpha * spmm_adj(h) + (1.0 - alpha) * spmm_re(h)

    h1 = jax.nn.relu(agg_new(features1) @ W1 + b1)
    out = agg_new(h1) @ W2 + b2
    return out


# speedup vs baseline: 2.0610x; 2.0493x over previous
import functools

import jax
import jax.numpy as jnp
from jax.experimental import pallas as pl
from jax.experimental.pallas import tpu as pltpu

N_BLK = 256


def _prompt_body(f_ref, pt_ref, gt_ref, pre_ref, cw_ref, o_ref):
    f = f_ref[...]
    pt = pt_ref[...]
    x = jax.nn.relu(pt * f)
    x = gt_ref[...] * x
    x1 = pre_ref[...] * f
    hid = cw_ref[0, 0] * x + cw_ref[0, 1] * x1
    o_ref[...] = jnp.where(hid > 0, hid, jnp.exp(jnp.minimum(hid, 0.0)) - 1.0)


def _prompt_stage(features, pt, global_token, pre_token, combine_weight):
    n, h = features.shape
    grid = (pl.cdiv(n, N_BLK),)
    return pl.pallas_call(
        _prompt_body,
        grid=grid,
        in_specs=[
            pl.BlockSpec((N_BLK, h), lambda i: (i, 0)),
            pl.BlockSpec((1, h), lambda i: (0, 0)),
            pl.BlockSpec((1, h), lambda i: (0, 0)),
            pl.BlockSpec((1, h), lambda i: (0, 0)),
            pl.BlockSpec((1, 2), lambda i: (0, 0), memory_space=pltpu.SMEM),
        ],
        out_specs=pl.BlockSpec((N_BLK, h), lambda i: (i, 0)),
        out_shape=jax.ShapeDtypeStruct((n, h), jnp.float32),
    )(features, pt, global_token, pre_token, combine_weight)


def _znorm_body(f1_ref, agg_ref, bt_ref, z_ref):
    r = jnp.concatenate([f1_ref[...], agg_ref[...]], axis=1) * bt_ref[...]
    nrm = jnp.sqrt(jnp.sum(r * r, axis=1, keepdims=True))
    z_ref[...] = r / (nrm + 1e-8)


def _znorm_stage(features1, agg, balance_token, n_pad):
    n, h = features1.shape
    grid = (pl.cdiv(n_pad, N_BLK),)
    return pl.pallas_call(
        _znorm_body,
        grid=grid,
        in_specs=[
            pl.BlockSpec((N_BLK, h), lambda i: (i, 0)),
            pl.BlockSpec((N_BLK, h), lambda i: (i, 0)),
            pl.BlockSpec((1, 2 * h), lambda i: (0, 0)),
        ],
        out_specs=pl.BlockSpec((N_BLK, 2 * h), lambda i: (i, 0)),
        out_shape=jax.ShapeDtypeStruct((n_pad, 2 * h), jnp.float32),
    )(features1, agg, balance_token)


def _simtopk_body(n_valid_ref, zb_ref, zall_ref, vals_ref, idx_ref, *, k, n_pad):
    rblk = zb_ref.shape[0]
    sim = jax.lax.dot_general(
        zb_ref[...], zall_ref[...], (((1,), (1,)), ((), ())),
        preferred_element_type=jnp.float32, precision=jax.lax.Precision.HIGHEST)
    ii = jax.lax.broadcasted_iota(jnp.int32, (rblk, n_pad), 1)
    n_valid = n_valid_ref[0]
    cur = jnp.where(ii < n_valid, sim, -jnp.inf)
    vals_l, idx_l = [], []
    for _ in range(k):
        m = jnp.max(cur, axis=1)
        eq = cur == m[:, None]
        am = jnp.min(jnp.where(eq, ii, n_pad), axis=1)
        vals_l.append(m)
        idx_l.append(am)
        cur = jnp.where(ii == am[:, None], -jnp.inf, cur)
    vals = jnp.stack(vals_l, axis=1)
    idx = jnp.stack(idx_l, axis=1)
    vals = jax.nn.relu(vals)
    vals = vals / (jnp.sum(vals, axis=1, keepdims=True) + 1e-8)
    vals_ref[...] = vals
    idx_ref[...] = idx


def _simtopk_stage(z, n_valid, k=10, rblk=256):
    n_pad, h2 = z.shape
    grid = (n_pad // rblk,)
    nv = jnp.full((1,), n_valid, dtype=jnp.int32)
    return pl.pallas_call(
        functools.partial(_simtopk_body, k=k, n_pad=n_pad),
        grid=grid,
        in_specs=[
            pl.BlockSpec(memory_space=pltpu.SMEM),
            pl.BlockSpec((rblk, h2), lambda i: (i, 0)),
            pl.BlockSpec((n_pad, h2), lambda i: (0, 0)),
        ],
        out_specs=[
            pl.BlockSpec((rblk, k), lambda i: (i, 0)),
            pl.BlockSpec((rblk, k), lambda i: (i, 0)),
        ],
        out_shape=[
            jax.ShapeDtypeStruct((n_pad, k), jnp.float32),
            jax.ShapeDtypeStruct((n_pad, k), jnp.int32),
        ],
    )(nv, z, z)


def kernel(features, adj_indices, adj_values, down_k, tokens, wp_weight,
           global_token, pre_token, combine_weight, balance_token,
           W1, b1, W2, b2):
    n = features.shape[0]
    src = adj_indices[0]
    dst = adj_indices[1]
    pt = wp_weight @ tokens
    features1 = _prompt_stage(features, pt, global_token, pre_token,
                              combine_weight)

    agg = jax.ops.segment_sum(adj_values[:, None] * features1[src], dst,
                              num_segments=n)
    n_pad = 10240
    f1_pad = jnp.pad(features1, ((0, n_pad - n), (0, 0)))
    agg_pad = jnp.pad(agg, ((0, n_pad - n), (0, 0)))
    z = _znorm_stage(f1_pad, agg_pad, balance_token, n_pad)
    vals, idx = _simtopk_stage(z, n)
    vals = vals[:n]
    idx = idx[:n]
    alpha = 0.5

    def spmm_adj(h):
        return jax.ops.segment_sum(adj_values[:, None] * h[src], dst,
                                   num_segments=n)

    def spmm_re(h):
        return jnp.sum(vals[:, :, None] * h[idx], axis=1)

    def agg_new(h):
        return alpha * spmm_adj(h) + (1.0 - alpha) * spmm_re(h)

    h1 = jax.nn.relu(agg_new(features1) @ W1 + b1)
    out = agg_new(h1) @ W2 + b2
    return out
